# Initial kernel scaffold; baseline (speedup 1.0000x reference)
import jax
import jax.numpy as jnp
from jax.experimental import pallas as pl


def kernel(locations, cls_preds, reg_preds, cness_preds, image_size):
    raise NotImplementedError("write your pallas kernel here")


# speedup vs baseline: 1.3178x; 1.3178x over previous
import jax
import jax.numpy as jnp
from jax.experimental import pallas as pl

_PRE_NMS_THRESH = 0.05
_PRE_NMS_TOP_N = 1000
_NMS_THRESH = 0.6
_POST_TOP_N = 100


def _score_kernel(cls_ref, cn_ref, out_ref):
    cls_s = jax.nn.sigmoid(cls_ref[0])
    cn_s = jax.nn.sigmoid(cn_ref[0])
    s = cls_s * cn_s
    out_ref[0] = jnp.where(cls_s > _PRE_NMS_THRESH, s, -1.0)


def _nms_kernel(d_ref, out_ref):
    x1 = d_ref[0, 0:1, :]
    y1 = d_ref[0, 1:2, :]
    x2 = d_ref[0, 2:3, :]
    y2 = d_ref[0, 3:4, :]
    ts = d_ref[0, 4:5, :]
    K = x1.shape[1]

    w = x2 - x1
    h = y2 - y1
    valid = (ts > 0.0) & (w >= 0.0) & (h >= 0.0)
    sc = jnp.sqrt(jnp.clip(ts, 1e-12))
    sc = jnp.where(valid, sc, 0.0)
    area = jnp.clip(w, 0.0) * jnp.clip(h, 0.0)
    col = jax.lax.broadcasted_iota(jnp.int32, (1, K), 1)
    neg = -1e30

    def body(i, keep):
        sel = col == i
        xi1 = jnp.max(jnp.where(sel, x1, neg))
        yi1 = jnp.max(jnp.where(sel, y1, neg))
        xi2 = jnp.max(jnp.where(sel, x2, neg))
        yi2 = jnp.max(jnp.where(sel, y2, neg))
        ai = jnp.max(jnp.where(sel, area, neg))
        ki = jnp.max(jnp.where(sel, keep, 0.0))
        iw = jnp.clip(jnp.minimum(xi2, x2) - jnp.maximum(xi1, x1), 0.0)
        ih = jnp.clip(jnp.minimum(yi2, y2) - jnp.maximum(yi1, y1), 0.0)
        inter = iw * ih
        union = ai + area - inter
        iou = inter / jnp.maximum(union, 1e-9)
        suppress = (ki > 0.0) & (iou > _NMS_THRESH) & (col > i)
        return jnp.where(suppress, 0.0, keep)

    keep = jax.lax.fori_loop(0, K, body, jnp.ones((1, K), jnp.float32))
    final = jnp.where((keep > 0.0) & valid & (sc >= _PRE_NMS_THRESH), sc, 0.0)
    out_ref[0, 0:1, :] = final


def kernel(locations, cls_preds, reg_preds, cness_preds, image_size):
    B, N, C = cls_preds.shape
    R = 2000

    flat = pl.pallas_call(
        _score_kernel,
        grid=(B, N // R),
        in_specs=[
            pl.BlockSpec((1, R, C), lambda b, i: (b, i, 0)),
            pl.BlockSpec((1, R, 1), lambda b, i: (b, i, 0)),
        ],
        out_specs=pl.BlockSpec((1, R, C), lambda b, i: (b, i, 0)),
        out_shape=jax.ShapeDtypeStruct((B, N, C), jnp.float32),
    )(cls_preds, cness_preds).reshape(B, N * C)

    top_scores, top_idx = jax.lax.top_k(flat, _PRE_NMS_TOP_N)
    loc_idx = top_idx // C
    labels = top_idx % C + 1

    locs = locations[loc_idx]
    regs = jnp.take_along_axis(reg_preds, loc_idx[..., None], axis=1)
    img = jnp.asarray(image_size, jnp.float32)
    x1 = jnp.clip(locs[..., 0] - regs[..., 0], 0.0, img)
    y1 = jnp.clip(locs[..., 1] - regs[..., 1], 0.0, img)
    x2 = jnp.clip(locs[..., 0] + regs[..., 2], 0.0, img)
    y2 = jnp.clip(locs[..., 1] + regs[..., 3], 0.0, img)
    z = jnp.zeros_like(x1)
    packed = jnp.stack([x1, y1, x2, y2, top_scores, z, z, z], axis=1)

    final = pl.pallas_call(
        _nms_kernel,
        grid=(B,),
        in_specs=[pl.BlockSpec((1, 8, _PRE_NMS_TOP_N), lambda b: (b, 0, 0))],
        out_specs=pl.BlockSpec((1, 1, _PRE_NMS_TOP_N), lambda b: (b, 0, 0)),
        out_shape=jax.ShapeDtypeStruct((B, 1, _PRE_NMS_TOP_N), jnp.float32),
    )(packed)[:, 0, :]

    out_scores, pick = jax.lax.top_k(final, _POST_TOP_N)
    boxes = jnp.stack([x1, y1, x2, y2], axis=-1)
    out_boxes = jnp.take_along_axis(boxes, pick[..., None], axis=1)
    out_labels = jnp.where(
        out_scores > 0.0, jnp.take_along_axis(labels, pick, axis=1), 0
    )
    return out_boxes, out_scores, out_labels


# speedup vs baseline: 4.0488x; 3.0724x over previous
import jax
import jax.numpy as jnp
from jax.experimental import pallas as pl

_PRE_NMS_THRESH = 0.05
_PRE_NMS_TOP_N = 1000
_NMS_THRESH = 0.6
_POST_TOP_N = 100


def _score_kernel(cls_ref, cn_ref, out_ref):
    cls_s = jax.nn.sigmoid(cls_ref[0])
    cn_s = jax.nn.sigmoid(cn_ref[0])
    s = cls_s * cn_s
    out_ref[0] = jnp.where(cls_s > _PRE_NMS_THRESH, s, -1.0)


def _nms_kernel(d_ref, out_ref):
    x1 = d_ref[0]
    y1 = d_ref[1]
    x2 = d_ref[2]
    y2 = d_ref[3]
    ts = d_ref[4]
    K = x1.shape[1]

    w = x2 - x1
    h = y2 - y1
    valid = (ts > 0.0) & (w >= 0.0) & (h >= 0.0)
    sc = jnp.sqrt(jnp.clip(ts, 1e-12))
    sc = jnp.where(valid, sc, 0.0)
    area = jnp.clip(w, 0.0) * jnp.clip(h, 0.0)
    col = jax.lax.broadcasted_iota(jnp.int32, (1, K), 1)
    neg = -1e30

    def body(i, keep):
        sel = col == i
        xi1 = jnp.max(jnp.where(sel, x1, neg), axis=1, keepdims=True)
        yi1 = jnp.max(jnp.where(sel, y1, neg), axis=1, keepdims=True)
        xi2 = jnp.max(jnp.where(sel, x2, neg), axis=1, keepdims=True)
        yi2 = jnp.max(jnp.where(sel, y2, neg), axis=1, keepdims=True)
        ai = jnp.max(jnp.where(sel, area, neg), axis=1, keepdims=True)
        ki = jnp.max(jnp.where(sel, keep, 0.0), axis=1, keepdims=True)
        iw = jnp.clip(jnp.minimum(xi2, x2) - jnp.maximum(xi1, x1), 0.0)
        ih = jnp.clip(jnp.minimum(yi2, y2) - jnp.maximum(yi1, y1), 0.0)
        inter = iw * ih
        union = ai + area - inter
        iou = inter / jnp.maximum(union, 1e-9)
        suppress = (ki > 0.0) & (iou > _NMS_THRESH) & (col > i)
        return jnp.where(suppress, 0.0, keep)

    keep = jax.lax.fori_loop(0, K, body, jnp.ones_like(ts))
    out_ref[0] = jnp.where(
        (keep > 0.0) & valid & (sc >= _PRE_NMS_THRESH), sc, 0.0
    )


def kernel(locations, cls_preds, reg_preds, cness_preds, image_size):
    B, N, C = cls_preds.shape
    R = 2000

    scores3d = pl.pallas_call(
        _score_kernel,
        grid=(B, N // R),
        in_specs=[
            pl.BlockSpec((1, R, C), lambda b, i: (b, i, 0)),
            pl.BlockSpec((1, R, 1), lambda b, i: (b, i, 0)),
        ],
        out_specs=pl.BlockSpec((1, R, C), lambda b, i: (b, i, 0)),
        out_shape=jax.ShapeDtypeStruct((B, N, C), jnp.float32),
    )(cls_preds, cness_preds)

    loc_max = jnp.max(scores3d, axis=2)
    _, loc_sel = jax.lax.top_k(loc_max, _PRE_NMS_TOP_N)
    rows = jnp.take_along_axis(scores3d, loc_sel[..., None], axis=1)
    top_scores, g = jax.lax.top_k(rows.reshape(B, -1), _PRE_NMS_TOP_N)
    loc_idx = jnp.take_along_axis(loc_sel, g // C, axis=1)
    labels = g % C + 1

    locs = locations[loc_idx]
    regs = jnp.take_along_axis(reg_preds, loc_idx[..., None], axis=1)
    img = jnp.asarray(image_size, jnp.float32)
    x1 = jnp.clip(locs[..., 0] - regs[..., 0], 0.0, img)
    y1 = jnp.clip(locs[..., 1] - regs[..., 1], 0.0, img)
    x2 = jnp.clip(locs[..., 0] + regs[..., 2], 0.0, img)
    y2 = jnp.clip(locs[..., 1] + regs[..., 3], 0.0, img)
    z = jnp.zeros_like(x1)
    packed = jnp.stack([x1, y1, x2, y2, top_scores, z, z, z], axis=0)

    final = pl.pallas_call(
        _nms_kernel,
        grid=(1,),
        in_specs=[pl.BlockSpec((8, B, _PRE_NMS_TOP_N), lambda i: (0, 0, 0))],
        out_specs=pl.BlockSpec((1, B, _PRE_NMS_TOP_N), lambda i: (0, 0, 0)),
        out_shape=jax.ShapeDtypeStruct((1, B, _PRE_NMS_TOP_N), jnp.float32),
    )(packed)[0]

    out_scores, pick = jax.lax.top_k(final, _POST_TOP_N)
    boxes = jnp.stack([x1, y1, x2, y2], axis=-1)
    out_boxes = jnp.take_along_axis(boxes, pick[..., None], axis=1)
    out_labels = jnp.where(
        out_scores > 0.0, jnp.take_along_axis(labels, pick, axis=1), 0
    )
    return out_boxes, out_scores, out_labels


# speedup vs baseline: 4.1785x; 1.0320x over previous
import jax
import jax.numpy as jnp
from jax.experimental import pallas as pl

_PRE_NMS_THRESH = 0.05
_PRE_NMS_TOP_N = 1000
_NMS_THRESH = 0.6
_POST_TOP_N = 100


def _score_kernel(cls_ref, cn_ref, out_ref):
    cls_s = jax.nn.sigmoid(cls_ref[0])
    cn_s = jax.nn.sigmoid(cn_ref[0])
    s = cls_s * cn_s
    out_ref[0] = jnp.where(cls_s > _PRE_NMS_THRESH, s, -1.0)


def _nms_kernel(d_ref, out_ref):
    x1 = d_ref[0]
    y1 = d_ref[1]
    x2 = d_ref[2]
    y2 = d_ref[3]
    ts = d_ref[4]
    K = x1.shape[1]

    w = x2 - x1
    h = y2 - y1
    valid = (ts > 0.0) & (w >= 0.0) & (h >= 0.0)
    sc = jnp.sqrt(jnp.clip(ts, 1e-12))
    sc = jnp.where(valid, sc, 0.0)
    area = jnp.clip(w, 0.0) * jnp.clip(h, 0.0)
    col = jax.lax.broadcasted_iota(jnp.int32, (1, K), 1)
    neg = -1e30

    def body(i, keep):
        sel = col == i
        xi1 = jnp.max(jnp.where(sel, x1, neg), axis=1, keepdims=True)
        yi1 = jnp.max(jnp.where(sel, y1, neg), axis=1, keepdims=True)
        xi2 = jnp.max(jnp.where(sel, x2, neg), axis=1, keepdims=True)
        yi2 = jnp.max(jnp.where(sel, y2, neg), axis=1, keepdims=True)
        ai = jnp.max(jnp.where(sel, area, neg), axis=1, keepdims=True)
        ki = jnp.max(jnp.where(sel, keep, 0.0), axis=1, keepdims=True)
        iw = jnp.clip(jnp.minimum(xi2, x2) - jnp.maximum(xi1, x1), 0.0)
        ih = jnp.clip(jnp.minimum(yi2, y2) - jnp.maximum(yi1, y1), 0.0)
        inter = iw * ih
        union = ai + area - inter
        iou = inter / jnp.maximum(union, 1e-9)
        suppress = (ki > 0.0) & (iou > _NMS_THRESH) & (col > i)
        return jnp.where(suppress, 0.0, keep)

    keep = jax.lax.fori_loop(0, K, body, jnp.ones_like(ts))
    out_ref[0] = jnp.where(
        (keep > 0.0) & valid & (sc >= _PRE_NMS_THRESH), sc, 0.0
    )


def kernel(locations, cls_preds, reg_preds, cness_preds, image_size):
    B, N, C = cls_preds.shape
    R = 2000

    scores3d = pl.pallas_call(
        _score_kernel,
        grid=(B, N // R),
        in_specs=[
            pl.BlockSpec((1, R, C), lambda b, i: (b, i, 0)),
            pl.BlockSpec((1, R, 1), lambda b, i: (b, i, 0)),
        ],
        out_specs=pl.BlockSpec((1, R, C), lambda b, i: (b, i, 0)),
        out_shape=jax.ShapeDtypeStruct((B, N, C), jnp.float32),
    )(cls_preds, cness_preds)

    loc_max = jnp.max(scores3d, axis=2)
    _, loc_sel = jax.lax.approx_max_k(loc_max, _PRE_NMS_TOP_N, recall_target=1.0)
    rows = jnp.take_along_axis(scores3d, loc_sel[..., None], axis=1)
    top_scores, g = jax.lax.approx_max_k(rows.reshape(B, -1), _PRE_NMS_TOP_N, recall_target=1.0)
    loc_idx = jnp.take_along_axis(loc_sel, g // C, axis=1)
    labels = g % C + 1

    locs = locations[loc_idx]
    regs = jnp.take_along_axis(reg_preds, loc_idx[..., None], axis=1)
    img = jnp.asarray(image_size, jnp.float32)
    x1 = jnp.clip(locs[..., 0] - regs[..., 0], 0.0, img)
    y1 = jnp.clip(locs[..., 1] - regs[..., 1], 0.0, img)
    x2 = jnp.clip(locs[..., 0] + regs[..., 2], 0.0, img)
    y2 = jnp.clip(locs[..., 1] + regs[..., 3], 0.0, img)
    z = jnp.zeros_like(x1)
    packed = jnp.stack([x1, y1, x2, y2, top_scores, z, z, z], axis=0)

    final = pl.pallas_call(
        _nms_kernel,
        grid=(1,),
        in_specs=[pl.BlockSpec((8, B, _PRE_NMS_TOP_N), lambda i: (0, 0, 0))],
        out_specs=pl.BlockSpec((1, B, _PRE_NMS_TOP_N), lambda i: (0, 0, 0)),
        out_shape=jax.ShapeDtypeStruct((1, B, _PRE_NMS_TOP_N), jnp.float32),
    )(packed)[0]

    out_scores, pick = jax.lax.top_k(final, _POST_TOP_N)
    boxes = jnp.stack([x1, y1, x2, y2], axis=-1)
    out_boxes = jnp.take_along_axis(boxes, pick[..., None], axis=1)
    out_labels = jnp.where(
        out_scores > 0.0, jnp.take_along_axis(labels, pick, axis=1), 0
    )
    return out_boxes, out_scores, out_labels
